# Initial kernel scaffold; baseline (speedup 1.0000x reference)
import jax
import jax.numpy as jnp
from jax.experimental import pallas as pl


def kernel(users, items, he_edge_index, he_values, ho_edge_index, ho_values, degree_he, user_emb_w, item_emb_w):
    raise NotImplementedError("write your pallas kernel here")


# speedup vs baseline: 2.0507x; 2.0507x over previous
import functools

import jax
import jax.numpy as jnp
from jax import lax
from jax.experimental import pallas as pl
from jax.experimental.pallas import tpu as pltpu
from jax.experimental.pallas import tpu_sc as plsc

NUM_USERS = 20000
NUM_ITEMS = 30000
NN = NUM_USERS + NUM_ITEMS
NNP = 50048
D = 64
DH = 32
NL = 3
NE = 800000
B = 4096

NTILES = 16
E = 80
EPT = NE // NTILES
NJ = EPT // E
RPT = NNP // NTILES
ZR = 391

_mesh = plsc.VectorSubcoreMesh(core_axis_name="c", subcore_axis_name="s")


@functools.partial(
    pl.kernel,
    out_type=(
        jax.ShapeDtypeStruct((2, NNP, DH), jnp.float32),
        jax.ShapeDtypeStruct((2, NNP, DH), jnp.float32),
    ),
    mesh=_mesh,
    scratch_types=[
        pltpu.VMEM((E,), jnp.int32),
        pltpu.VMEM((E,), jnp.int32),
        pltpu.VMEM((E,), jnp.float32),
        pltpu.VMEM((E, DH), jnp.float32),
        pltpu.VMEM((ZR, DH), jnp.float32),
        pltpu.VMEM_SHARED((NNP, DH), jnp.float32),
        pltpu.SemaphoreType.DMA,
    ],
    compiler_params=pltpu.CompilerParams(use_tc_tiling_on_sc=False),
)
def _spmm2(x2, rows_ho, cols_ho, vals_ho, rows_he, cols_he, vals_he, zrows,
           out_ho, out_he, idx_r, idx_c, vv, rbuf, zbuf, acc, sem):
    cid = lax.axis_index("c")
    sid = lax.axis_index("s")
    coff = cid * NNP
    pltpu.sync_copy(zrows, zbuf)

    for rows_h, cols_h, vals_h, out_h in (
        (rows_ho, cols_ho, vals_ho, out_ho),
        (rows_he, cols_he, vals_he, out_he),
    ):
        for k in range(RPT // ZR):
            pltpu.sync_copy(zbuf, acc.at[pl.ds(sid * RPT + k * ZR, ZR)])
        plsc.subcore_barrier()

        def chunk_body(j, _, rows_h=rows_h, cols_h=cols_h, vals_h=vals_h):
            off = sid * EPT + j * E
            pltpu.sync_copy(rows_h.at[pl.ds(off, E)], idx_r)
            pltpu.sync_copy(cols_h.at[pl.ds(off, E)], idx_c)
            pltpu.sync_copy(vals_h.at[pl.ds(off, E)], vv)
            for i in range(E // 16):
                idx_c[pl.ds(i * 16, 16)] = idx_c[pl.ds(i * 16, 16)] + coff
            pltpu.async_copy(x2.at[idx_c], rbuf, sem).wait()

            def scale_body(g, _):
                v16 = vv[pl.ds(g * 16, 16)]
                base = g * 16
                for l in range(16):
                    e = base + l
                    v = v16[l]
                    rbuf[e, pl.ds(0, 16)] = rbuf[e, pl.ds(0, 16)] * v
                    rbuf[e, pl.ds(16, 16)] = rbuf[e, pl.ds(16, 16)] * v
                return 0

            lax.fori_loop(0, E // 16, scale_body, 0)
            pltpu.sync_copy(rbuf, acc.at[idx_r], add=True)
            return 0

        lax.fori_loop(0, NJ, chunk_body, 0)
        plsc.subcore_barrier()
        pltpu.sync_copy(acc.at[pl.ds(sid * RPT, RPT)],
                        out_h.at[cid, pl.ds(sid * RPT, RPT)])
        plsc.subcore_barrier()


BN = 3128


def _update_body(ho_ref, he_ref, whe_ref, acc_ref, x_out, acc_out, w_out):
    ho0 = ho_ref[0]
    ho1 = ho_ref[1]
    he0 = he_ref[0]
    he1 = he_ref[1]
    whe = whe_ref[...]
    who = 1.0 - whe
    a0 = who * ho0 + whe * he0
    a1 = who * ho1 + whe * he1
    t_ho = jnp.sum(a0 * ho0 + a1 * ho1, axis=1, keepdims=True)
    t_he = jnp.sum(a0 * he0 + a1 * he1, axis=1, keepdims=True)
    who2 = who + 0.1 * t_ho
    whe2 = whe + 0.1 * t_he
    who3 = who2 / (who2 + whe2)
    w_out[...] = 1.0 - who3
    x_out[0] = a0
    x_out[1] = a1
    acc_out[0] = acc_ref[0] + a0
    acc_out[1] = acc_ref[1] + a1


def _update(ho2, he2, whe, acc2):
    big = pl.BlockSpec((2, BN, DH), lambda i: (0, i, 0))
    small = pl.BlockSpec((BN, 1), lambda i: (i, 0))
    return pl.pallas_call(
        _update_body,
        grid=(NNP // BN,),
        in_specs=[big, big, small, big],
        out_specs=[big, big, small],
        out_shape=[
            jax.ShapeDtypeStruct((2, NNP, DH), jnp.float32),
            jax.ShapeDtypeStruct((2, NNP, DH), jnp.float32),
            jax.ShapeDtypeStruct((NNP, 1), jnp.float32),
        ],
    )(ho2, he2, whe, acc2)


BPT = B // NTILES
BC = 128


@functools.partial(
    pl.kernel,
    out_type=(
        jax.ShapeDtypeStruct((2, B, DH), jnp.float32),
        jax.ShapeDtypeStruct((2, B, DH), jnp.float32),
    ),
    mesh=_mesh,
    scratch_types=[
        pltpu.VMEM((BC,), jnp.int32),
        pltpu.VMEM((BC,), jnp.int32),
        pltpu.VMEM((BC, DH), jnp.float32),
        pltpu.VMEM((BC, DH), jnp.float32),
        pltpu.SemaphoreType.DMA,
    ],
    compiler_params=pltpu.CompilerParams(use_tc_tiling_on_sc=False),
)
def _gather_rows(accflat, users, items, u_out, i_out, iu, ii, bu, bi, sem):
    cid = lax.axis_index("c")
    sid = lax.axis_index("s")
    coff = cid * NNP
    for q in range(BPT // BC):
        off = sid * BPT + q * BC
        pltpu.sync_copy(users.at[pl.ds(off, BC)], iu)
        pltpu.sync_copy(items.at[pl.ds(off, BC)], ii)
        for i in range(BC // 16):
            iu[pl.ds(i * 16, 16)] = iu[pl.ds(i * 16, 16)] + coff
            ii[pl.ds(i * 16, 16)] = ii[pl.ds(i * 16, 16)] + (coff + NUM_USERS)
        pltpu.async_copy(accflat.at[iu], bu, sem).wait()
        pltpu.async_copy(accflat.at[ii], bi, sem).wait()
        pltpu.sync_copy(bu, u_out.at[cid, pl.ds(off, BC)])
        pltpu.sync_copy(bi, i_out.at[cid, pl.ds(off, BC)])


def _combine_body(u_ref, i_ref, out_ref):
    s = u_ref[0] * i_ref[0] + u_ref[1] * i_ref[1]
    out_ref[...] = jnp.sum(s, axis=1) * 0.0625


def _combine(u2, i2):
    return pl.pallas_call(
        _combine_body,
        out_shape=jax.ShapeDtypeStruct((B,), jnp.float32),
    )(u2, i2)


def kernel(users, items, he_edge_index, he_values, ho_edge_index, ho_values,
           degree_he, user_emb_w, item_emb_w):
    all0 = jnp.concatenate([user_emb_w, item_emb_w], axis=0)
    all0 = jnp.pad(all0, ((0, NNP - NN), (0, 0)))
    x2 = jnp.concatenate([all0[:, :DH], all0[:, DH:]], axis=0)
    acc2 = jnp.reshape(x2, (2, NNP, DH))
    whe = jnp.pad(degree_he, ((0, NNP - NN), (0, 0)))
    zrows = jnp.zeros((ZR, DH), jnp.float32)

    rows_he, cols_he = he_edge_index[0], he_edge_index[1]
    vals_he = he_values
    rows_ho, cols_ho = ho_edge_index[0], ho_edge_index[1]
    vals_ho = ho_values

    for _ in range(NL):
        ho2, he2 = _spmm2(x2, rows_ho, cols_ho, vals_ho,
                          rows_he, cols_he, vals_he, zrows)
        x2_next, acc2, whe = _update(ho2, he2, whe, acc2)
        x2 = jnp.reshape(x2_next, (2 * NNP, DH))

    accflat = jnp.reshape(acc2, (2 * NNP, DH))
    u2, i2 = _gather_rows(accflat, users, items)
    return _combine(u2, i2)


# speedup vs baseline: 8.5341x; 4.1616x over previous
import functools

import jax
import jax.numpy as jnp
from jax import lax
from jax.experimental import pallas as pl
from jax.experimental.pallas import tpu as pltpu
from jax.experimental.pallas import tpu_sc as plsc

NUM_USERS = 20000
NUM_ITEMS = 30000
NN = NUM_USERS + NUM_ITEMS
NNP = 50048
D = 64
DH = 32
NL = 3
NE = 800000
B = 4096

NTILES = 16
E = 80
KCH = 5
SCH = KCH * E
EPT = NE // NTILES
NT = EPT // SCH
RPT = NNP // NTILES


_mesh = plsc.VectorSubcoreMesh(core_axis_name="c", subcore_axis_name="s")


@functools.partial(
    pl.kernel,
    out_type=(
        jax.ShapeDtypeStruct((2, NNP, DH), jnp.float32),
        jax.ShapeDtypeStruct((2, NNP, DH), jnp.float32),
    ),
    mesh=_mesh,
    scratch_types=[
        pltpu.VMEM((2, KCH, E), jnp.int32),
        pltpu.VMEM((2, KCH, E), jnp.int32),
        pltpu.VMEM((2, SCH), jnp.float32),
        pltpu.VMEM((2, SCH, DH), jnp.float32),
        pltpu.VMEM_SHARED((NNP, DH), jnp.float32),
        pltpu.SemaphoreType.DMA,
        pltpu.SemaphoreType.DMA,
        pltpu.SemaphoreType.DMA,
        pltpu.SemaphoreType.DMA,
        pltpu.SemaphoreType.DMA,
        pltpu.SemaphoreType.DMA,
    ],
    compiler_params=pltpu.CompilerParams(use_tc_tiling_on_sc=False),
)
def _spmm2(x2, rows_ho, cols_ho, vals_ho, rows_he, cols_he, vals_he, zrows,
           out_ho, out_he, idxr, idxc, vv, rbuf, acc,
           semi0, semi1, semg0, semg1, sems0, sems1):
    cid = lax.axis_index("c")
    sid = lax.axis_index("s")
    coff = cid * NNP
    semi = (semi0, semi1)
    semg = (semg0, semg1)
    sems = (sems0, sems1)

    for rows2d, cols2d, vals_h, out_h in (
        (rows_ho, cols_ho, vals_ho, out_ho),
        (rows_he, cols_he, vals_he, out_he),
    ):
        base_row = sid * (EPT // E)
        base_e = sid * EPT

        def fire_idx(t, p, rows2d=rows2d, cols2d=cols2d, vals_h=vals_h):
            pltpu.async_copy(rows2d.at[pl.ds(base_row + t * KCH, KCH)],
                             idxr.at[p], semi[p])
            pltpu.async_copy(cols2d.at[pl.ds(base_row + t * KCH, KCH)],
                             idxc.at[p], semi[p])
            pltpu.async_copy(vals_h.at[pl.ds(base_e + t * SCH, SCH)],
                             vv.at[p], semi[p])

        def wait_idx(t, p, rows2d=rows2d, cols2d=cols2d, vals_h=vals_h):
            pltpu.make_async_copy(rows2d.at[pl.ds(base_row + t * KCH, KCH)],
                                  idxr.at[p], semi[p]).wait()
            pltpu.make_async_copy(cols2d.at[pl.ds(base_row + t * KCH, KCH)],
                                  idxc.at[p], semi[p]).wait()
            pltpu.make_async_copy(vals_h.at[pl.ds(base_e + t * SCH, SCH)],
                                  vv.at[p], semi[p]).wait()

        def prep(t, p):
            wait_idx(t, p)
            for k in range(KCH):
                for i in range(E // 16):
                    idxc[p, k, pl.ds(i * 16, 16)] = (
                        idxc[p, k, pl.ds(i * 16, 16)] + coff)
            for k in range(KCH):
                pltpu.async_copy(x2.at[idxc.at[p, k]],
                                 rbuf.at[p, pl.ds(k * E, E)], semg[p])

        def process(t, p):
            for k in range(KCH):
                pltpu.make_async_copy(x2.at[idxc.at[p, k]],
                                      rbuf.at[p, pl.ds(k * E, E)],
                                      semg[p]).wait()

            def scale_body(g, _):
                v16 = vv[p, pl.ds(g * 16, 16)]
                base = g * 16
                for l in range(16):
                    e = base + l
                    v = v16[l]
                    rbuf[p, e, pl.ds(0, 16)] = rbuf[p, e, pl.ds(0, 16)] * v
                    rbuf[p, e, pl.ds(16, 16)] = rbuf[p, e, pl.ds(16, 16)] * v
                return 0

            lax.fori_loop(0, SCH // 16, scale_body, 0)
            for k in range(KCH):
                pltpu.async_copy(rbuf.at[p, pl.ds(k * E, E)],
                                 acc.at[idxr.at[p, k]], sems[p], add=True)

        def drain_scatter(p):
            for k in range(KCH):
                pltpu.make_async_copy(rbuf.at[p, pl.ds(k * E, E)],
                                      acc.at[idxr.at[p, k]], sems[p]).wait()

        def body(t, p):
            q = 1 - p
            prep(t + 1, q)
            process(t, p)
            drain_scatter(p)
            t2 = t + 2

            @pl.when(t2 < NT)
            def _():
                fire_idx(t2, p)

        fire_idx(0, 0)
        pltpu.sync_copy(zrows, acc.at[pl.ds(sid * RPT, RPT)])
        plsc.subcore_barrier()
        prep(0, 0)
        fire_idx(1, 1)

        def loop_body(jj, _):
            t = 2 * jj
            body(t, 0)
            body(t + 1, 1)
            return 0

        lax.fori_loop(0, (NT - 1) // 2, loop_body, 0)
        process(NT - 1, (NT - 1) % 2)
        drain_scatter((NT - 1) % 2)
        plsc.subcore_barrier()
        pltpu.sync_copy(acc.at[pl.ds(sid * RPT, RPT)],
                        out_h.at[cid, pl.ds(sid * RPT, RPT)])
        plsc.subcore_barrier()


BN = 3128


def _update_body(ho_ref, he_ref, whe_ref, acc_ref, x_out, acc_out, w_out):
    ho0 = ho_ref[0]
    ho1 = ho_ref[1]
    he0 = he_ref[0]
    he1 = he_ref[1]
    whe = whe_ref[...]
    who = 1.0 - whe
    a0 = who * ho0 + whe * he0
    a1 = who * ho1 + whe * he1
    t_ho = jnp.sum(a0 * ho0 + a1 * ho1, axis=1, keepdims=True)
    t_he = jnp.sum(a0 * he0 + a1 * he1, axis=1, keepdims=True)
    who2 = who + 0.1 * t_ho
    whe2 = whe + 0.1 * t_he
    who3 = who2 / (who2 + whe2)
    w_out[...] = 1.0 - who3
    x_out[0] = a0
    x_out[1] = a1
    acc_out[0] = acc_ref[0] + a0
    acc_out[1] = acc_ref[1] + a1


def _update(ho2, he2, whe, acc2):
    big = pl.BlockSpec((2, BN, DH), lambda i: (0, i, 0))
    small = pl.BlockSpec((BN, 1), lambda i: (i, 0))
    return pl.pallas_call(
        _update_body,
        grid=(NNP // BN,),
        in_specs=[big, big, small, big],
        out_specs=[big, big, small],
        out_shape=[
            jax.ShapeDtypeStruct((2, NNP, DH), jnp.float32),
            jax.ShapeDtypeStruct((2, NNP, DH), jnp.float32),
            jax.ShapeDtypeStruct((NNP, 1), jnp.float32),
        ],
    )(ho2, he2, whe, acc2)


BPT = B // NTILES
BC = 128


@functools.partial(
    pl.kernel,
    out_type=(
        jax.ShapeDtypeStruct((2, B, DH), jnp.float32),
        jax.ShapeDtypeStruct((2, B, DH), jnp.float32),
    ),
    mesh=_mesh,
    scratch_types=[
        pltpu.VMEM((BC,), jnp.int32),
        pltpu.VMEM((BC,), jnp.int32),
        pltpu.VMEM((BC, DH), jnp.float32),
        pltpu.VMEM((BC, DH), jnp.float32),
        pltpu.SemaphoreType.DMA,
    ],
    compiler_params=pltpu.CompilerParams(use_tc_tiling_on_sc=False),
)
def _gather_rows(accflat, users, items, u_out, i_out, iu, ii, bu, bi, sem):
    cid = lax.axis_index("c")
    sid = lax.axis_index("s")
    coff = cid * NNP
    for q in range(BPT // BC):
        off = sid * BPT + q * BC
        pltpu.sync_copy(users.at[pl.ds(off, BC)], iu)
        pltpu.sync_copy(items.at[pl.ds(off, BC)], ii)
        for i in range(BC // 16):
            iu[pl.ds(i * 16, 16)] = iu[pl.ds(i * 16, 16)] + coff
            ii[pl.ds(i * 16, 16)] = ii[pl.ds(i * 16, 16)] + (coff + NUM_USERS)
        pltpu.async_copy(accflat.at[iu], bu, sem).wait()
        pltpu.async_copy(accflat.at[ii], bi, sem).wait()
        pltpu.sync_copy(bu, u_out.at[cid, pl.ds(off, BC)])
        pltpu.sync_copy(bi, i_out.at[cid, pl.ds(off, BC)])


def _combine_body(u_ref, i_ref, out_ref):
    s = u_ref[0] * i_ref[0] + u_ref[1] * i_ref[1]
    out_ref[...] = jnp.sum(s, axis=1) * 0.0625


def _combine(u2, i2):
    return pl.pallas_call(
        _combine_body,
        out_shape=jax.ShapeDtypeStruct((B,), jnp.float32),
    )(u2, i2)


def kernel(users, items, he_edge_index, he_values, ho_edge_index, ho_values,
           degree_he, user_emb_w, item_emb_w):
    all0 = jnp.concatenate([user_emb_w, item_emb_w], axis=0)
    all0 = jnp.pad(all0, ((0, NNP - NN), (0, 0)))
    x2 = jnp.concatenate([all0[:, :DH], all0[:, DH:]], axis=0)
    acc2 = jnp.reshape(x2, (2, NNP, DH))
    whe = jnp.pad(degree_he, ((0, NNP - NN), (0, 0)))
    zrows = jnp.zeros((RPT, DH), jnp.float32)

    rows_he = jnp.reshape(he_edge_index[0], (NE // E, E))
    cols_he = jnp.reshape(he_edge_index[1], (NE // E, E))
    vals_he = he_values
    rows_ho = jnp.reshape(ho_edge_index[0], (NE // E, E))
    cols_ho = jnp.reshape(ho_edge_index[1], (NE // E, E))
    vals_ho = ho_values

    for _ in range(NL):
        ho2, he2 = _spmm2(x2, rows_ho, cols_ho, vals_ho,
                          rows_he, cols_he, vals_he, zrows)
        x2_next, acc2, whe = _update(ho2, he2, whe, acc2)
        x2 = jnp.reshape(x2_next, (2 * NNP, DH))

    accflat = jnp.reshape(acc2, (2 * NNP, DH))
    u2, i2 = _gather_rows(accflat, users, items)
    return _combine(u2, i2)


# speedup vs baseline: 9.7347x; 1.1407x over previous
import functools

import jax
import jax.numpy as jnp
from jax import lax
from jax.experimental import pallas as pl
from jax.experimental.pallas import tpu as pltpu
from jax.experimental.pallas import tpu_sc as plsc

NUM_USERS = 20000
NUM_ITEMS = 30000
NN = NUM_USERS + NUM_ITEMS
NNP = 50048
D = 64
DH = 32
NL = 3
NE = 800000
B = 4096

NTILES = 16
E = 80
KCH = 5
SCH = KCH * E
EPT = NE // NTILES
NT = EPT // SCH
RPT = NNP // NTILES


_mesh = plsc.VectorSubcoreMesh(core_axis_name="c", subcore_axis_name="s")


@functools.partial(
    pl.kernel,
    out_type=jax.ShapeDtypeStruct((2, 2, NNP, DH), jnp.float32),
    mesh=_mesh,
    scratch_types=[
        pltpu.VMEM((3, KCH, E), jnp.int32),
        pltpu.VMEM((3, KCH, E), jnp.int32),
        pltpu.VMEM((2, SCH), jnp.float32),
        pltpu.VMEM((2, SCH, DH), jnp.float32),
        pltpu.VMEM_SHARED((NNP, DH), jnp.float32),
        pltpu.SemaphoreType.DMA,
        pltpu.SemaphoreType.DMA,
        pltpu.SemaphoreType.DMA,
        pltpu.SemaphoreType.DMA,
        pltpu.SemaphoreType.DMA,
        pltpu.SemaphoreType.DMA,
        pltpu.SemaphoreType.DMA,
    ],
    compiler_params=pltpu.CompilerParams(use_tc_tiling_on_sc=False),
)
def _spmm2(x2, rows_all, cols_all, vals_all, zrows, out4,
           idxr, idxc, vv, rbuf, acc,
           semi0, semi1, semi2, semg0, semg1, sems0, sems1):
    cid = lax.axis_index("c")
    sid = lax.axis_index("s")
    coff = cid * NNP
    semi = (semi0, semi1, semi2)
    semg = (semg0, semg1)
    sems = (sems0, sems1)

    def graph_body(g, _):
        base_row = g * (NE // E) + sid * (EPT // E)
        base_e = g * NE + sid * EPT

        def fire_idx(t, r, p):
            pltpu.async_copy(rows_all.at[pl.ds(base_row + t * KCH, KCH)],
                             idxr.at[r], semi[r])
            pltpu.async_copy(cols_all.at[pl.ds(base_row + t * KCH, KCH)],
                             idxc.at[r], semi[r])
            pltpu.async_copy(vals_all.at[pl.ds(base_e + t * SCH, SCH)],
                             vv.at[p], semi[r])

        def prep(t, p, r):
            pltpu.make_async_copy(rows_all.at[pl.ds(base_row + t * KCH, KCH)],
                                  idxr.at[r], semi[r]).wait()
            pltpu.make_async_copy(cols_all.at[pl.ds(base_row + t * KCH, KCH)],
                                  idxc.at[r], semi[r]).wait()
            pltpu.make_async_copy(vals_all.at[pl.ds(base_e + t * SCH, SCH)],
                                  vv.at[p], semi[r]).wait()

            def off_body(k, _):
                for i in range(E // 16):
                    idxc[r, k, pl.ds(i * 16, 16)] = (
                        idxc[r, k, pl.ds(i * 16, 16)] + coff)
                pltpu.async_copy(x2.at[idxc.at[r, k]],
                                 rbuf.at[p, pl.ds(k * E, E)], semg[p])
                return 0

            lax.fori_loop(0, KCH, off_body, 0)

        def process(t, p, r):
            def chunk_body(k, _):
                pltpu.make_async_copy(x2.at[idxc.at[r, k]],
                                      rbuf.at[p, pl.ds(k * E, E)],
                                      semg[p]).wait()

                def scale_body(gg, _):
                    base = k * E + gg * 16
                    v16 = vv[p, pl.ds(base, 16)]
                    for l in range(16):
                        e = base + l
                        v = v16[l]
                        rbuf[p, e, pl.ds(0, 16)] = rbuf[p, e, pl.ds(0, 16)] * v
                        rbuf[p, e, pl.ds(16, 16)] = (
                            rbuf[p, e, pl.ds(16, 16)] * v)
                    return 0

                lax.fori_loop(0, E // 16, scale_body, 0)
                pltpu.async_copy(rbuf.at[p, pl.ds(k * E, E)],
                                 acc.at[idxr.at[r, k]], sems[p], add=True)
                return 0

            lax.fori_loop(0, KCH, chunk_body, 0)

        def drain_scatter(p, r):
            def drain_body(k, _):
                pltpu.make_async_copy(rbuf.at[p, pl.ds(k * E, E)],
                                      acc.at[idxr.at[r, k]], sems[p]).wait()
                return 0

            lax.fori_loop(0, KCH, drain_body, 0)

        def body(t, p, r, first=False, tail=0):
            if not first:
                drain_scatter((p + 1) % 2, (r + 2) % 3)
            if tail < 2:
                prep(t + 1, (p + 1) % 2, (r + 1) % 3)
            process(t, p, r)
            if tail < 1:
                fire_idx(t + 2, (r + 2) % 3, p)

        fire_idx(0, 0, 0)
        pltpu.sync_copy(zrows, acc.at[pl.ds(sid * RPT, RPT)])
        plsc.subcore_barrier()
        prep(0, 0, 0)
        fire_idx(1, 1, 1)

        body(0, 0, 0, first=True)

        def loop_body(jj, _):
            t = 1 + 6 * jj
            for u in range(6):
                body(t + u, (1 + u) % 2, (1 + u) % 3)
            return 0

        lax.fori_loop(0, 20, loop_body, 0)
        body(121, 1, 1)
        body(122, 0, 2)
        body(123, 1, 0, tail=1)
        body(124, 0, 1, tail=2)
        drain_scatter(0, 1)
        plsc.subcore_barrier()
        pltpu.sync_copy(acc.at[pl.ds(sid * RPT, RPT)],
                        out4.at[g, cid, pl.ds(sid * RPT, RPT)])
        plsc.subcore_barrier()
        return 0

    lax.fori_loop(0, 2, graph_body, 0)


BN = 3128


def _update_body(oh_ref, whe_ref, acc_ref, x_out, acc_out, w_out):
    ho0 = oh_ref[0, 0]
    ho1 = oh_ref[0, 1]
    he0 = oh_ref[1, 0]
    he1 = oh_ref[1, 1]
    whe = whe_ref[...]
    who = 1.0 - whe
    a0 = who * ho0 + whe * he0
    a1 = who * ho1 + whe * he1
    t_ho = jnp.sum(a0 * ho0 + a1 * ho1, axis=1, keepdims=True)
    t_he = jnp.sum(a0 * he0 + a1 * he1, axis=1, keepdims=True)
    who2 = who + 0.1 * t_ho
    whe2 = whe + 0.1 * t_he
    who3 = who2 / (who2 + whe2)
    w_out[...] = 1.0 - who3
    x_out[0] = a0
    x_out[1] = a1
    acc_out[0] = acc_ref[0] + a0
    acc_out[1] = acc_ref[1] + a1


def _update(out4, whe, acc2):
    big = pl.BlockSpec((2, BN, DH), lambda i: (0, i, 0))
    big4 = pl.BlockSpec((2, 2, BN, DH), lambda i: (0, 0, i, 0))
    small = pl.BlockSpec((BN, 1), lambda i: (i, 0))
    return pl.pallas_call(
        _update_body,
        grid=(NNP // BN,),
        in_specs=[big4, small, big],
        out_specs=[big, big, small],
        out_shape=[
            jax.ShapeDtypeStruct((2, NNP, DH), jnp.float32),
            jax.ShapeDtypeStruct((2, NNP, DH), jnp.float32),
            jax.ShapeDtypeStruct((NNP, 1), jnp.float32),
        ],
    )(out4, whe, acc2)


BPT = B // NTILES
BC = 128


@functools.partial(
    pl.kernel,
    out_type=(
        jax.ShapeDtypeStruct((2, B, DH), jnp.float32),
        jax.ShapeDtypeStruct((2, B, DH), jnp.float32),
    ),
    mesh=_mesh,
    scratch_types=[
        pltpu.VMEM((BC,), jnp.int32),
        pltpu.VMEM((BC,), jnp.int32),
        pltpu.VMEM((BC, DH), jnp.float32),
        pltpu.VMEM((BC, DH), jnp.float32),
        pltpu.SemaphoreType.DMA,
    ],
    compiler_params=pltpu.CompilerParams(use_tc_tiling_on_sc=False),
)
def _gather_rows(accflat, users, items, u_out, i_out, iu, ii, bu, bi, sem):
    cid = lax.axis_index("c")
    sid = lax.axis_index("s")
    coff = cid * NNP
    for q in range(BPT // BC):
        off = sid * BPT + q * BC
        pltpu.sync_copy(users.at[pl.ds(off, BC)], iu)
        pltpu.sync_copy(items.at[pl.ds(off, BC)], ii)
        for i in range(BC // 16):
            iu[pl.ds(i * 16, 16)] = iu[pl.ds(i * 16, 16)] + coff
            ii[pl.ds(i * 16, 16)] = ii[pl.ds(i * 16, 16)] + (coff + NUM_USERS)
        pltpu.async_copy(accflat.at[iu], bu, sem).wait()
        pltpu.async_copy(accflat.at[ii], bi, sem).wait()
        pltpu.sync_copy(bu, u_out.at[cid, pl.ds(off, BC)])
        pltpu.sync_copy(bi, i_out.at[cid, pl.ds(off, BC)])


def _combine_body(u_ref, i_ref, out_ref):
    s = u_ref[0] * i_ref[0] + u_ref[1] * i_ref[1]
    out_ref[...] = jnp.sum(s, axis=1) * 0.0625


def _combine(u2, i2):
    return pl.pallas_call(
        _combine_body,
        out_shape=jax.ShapeDtypeStruct((B,), jnp.float32),
    )(u2, i2)


def kernel(users, items, he_edge_index, he_values, ho_edge_index, ho_values,
           degree_he, user_emb_w, item_emb_w):
    all0 = jnp.concatenate([user_emb_w, item_emb_w], axis=0)
    all0 = jnp.pad(all0, ((0, NNP - NN), (0, 0)))
    x2 = jnp.concatenate([all0[:, :DH], all0[:, DH:]], axis=0)
    acc2 = jnp.reshape(x2, (2, NNP, DH))
    whe = jnp.pad(degree_he, ((0, NNP - NN), (0, 0)))
    zrows = jnp.zeros((RPT, DH), jnp.float32)

    rows_all = jnp.reshape(
        jnp.concatenate([ho_edge_index[0], he_edge_index[0]]), (2 * NE // E, E))
    cols_all = jnp.reshape(
        jnp.concatenate([ho_edge_index[1], he_edge_index[1]]), (2 * NE // E, E))
    vals_all = jnp.concatenate([ho_values, he_values])

    for _ in range(NL):
        out4 = _spmm2(x2, rows_all, cols_all, vals_all, zrows)
        x2_next, acc2, whe = _update(out4, whe, acc2)
        x2 = jnp.reshape(x2_next, (2 * NNP, DH))

    accflat = jnp.reshape(acc2, (2 * NNP, DH))
    u2, i2 = _gather_rows(accflat, users, items)
    return _combine(u2, i2)


# speedup vs baseline: 13.0912x; 1.3448x over previous
import functools

import jax
import jax.numpy as jnp
from jax import lax
from jax.experimental import pallas as pl
from jax.experimental.pallas import tpu as pltpu
from jax.experimental.pallas import tpu_sc as plsc

NUM_USERS = 20000
NUM_ITEMS = 30000
NN = NUM_USERS + NUM_ITEMS
NNP = 50048
D = 64
DH = 32
NL = 3
NE = 800000
B = 4096

NTILES = 16
E = 80
KCH = 5
SCH = KCH * E
EPT = NE // NTILES
NT = EPT // SCH
RPT = NNP // NTILES


_mesh = plsc.VectorSubcoreMesh(core_axis_name="c", subcore_axis_name="s")


@functools.partial(
    pl.kernel,
    out_type=jax.ShapeDtypeStruct((2, 2, NNP, DH), jnp.float32),
    mesh=_mesh,
    scratch_types=[
        pltpu.VMEM((3, KCH, E), jnp.int32),
        pltpu.VMEM((3, KCH, E), jnp.int32),
        pltpu.VMEM((2, SCH), jnp.float32),
        pltpu.VMEM((2, SCH, DH), jnp.float32),
        pltpu.VMEM_SHARED((NNP, DH), jnp.float32),
        pltpu.SemaphoreType.DMA,
        pltpu.SemaphoreType.DMA,
        pltpu.SemaphoreType.DMA,
        pltpu.SemaphoreType.DMA,
        pltpu.SemaphoreType.DMA,
        pltpu.SemaphoreType.DMA,
        pltpu.SemaphoreType.DMA,
    ],
    compiler_params=pltpu.CompilerParams(use_tc_tiling_on_sc=False),
)
def _spmm2(x2, rows_all, cols_all, vals_all, zrows, out4,
           idxr, idxc, vv, rbuf, acc,
           semi0, semi1, semi2, semg0, semg1, sems0, sems1):
    cid = lax.axis_index("c")
    sid = lax.axis_index("s")
    coff = cid * NNP
    semi = (semi0, semi1, semi2)
    semg = (semg0, semg1)
    sems = (sems0, sems1)

    def graph_body(g, _):
        base_row = g * (NE // E) + sid * (EPT // E)
        base_e = g * NE + sid * EPT

        def fire_idx(t, r, p):
            pltpu.async_copy(rows_all.at[pl.ds(base_row + t * KCH, KCH)],
                             idxr.at[r], semi[r])
            pltpu.async_copy(cols_all.at[pl.ds(base_row + t * KCH, KCH)],
                             idxc.at[r], semi[r])
            pltpu.async_copy(vals_all.at[pl.ds(base_e + t * SCH, SCH)],
                             vv.at[p], semi[r])

        def prep(t, p, r):
            pltpu.make_async_copy(rows_all.at[pl.ds(base_row + t * KCH, KCH)],
                                  idxr.at[r], semi[r]).wait()
            pltpu.make_async_copy(cols_all.at[pl.ds(base_row + t * KCH, KCH)],
                                  idxc.at[r], semi[r]).wait()
            pltpu.make_async_copy(vals_all.at[pl.ds(base_e + t * SCH, SCH)],
                                  vv.at[p], semi[r]).wait()

            def off_body(k, _):
                for i in range(E // 16):
                    idxc[r, k, pl.ds(i * 16, 16)] = (
                        idxc[r, k, pl.ds(i * 16, 16)] + coff)
                pltpu.async_copy(x2.at[idxc.at[r, k]],
                                 rbuf.at[p, pl.ds(k * E, E)], semg[p])
                return 0

            lax.fori_loop(0, KCH, off_body, 0)

        def process(t, p, r):
            def chunk_body(k, _):
                pltpu.make_async_copy(x2.at[idxc.at[r, k]],
                                      rbuf.at[p, pl.ds(k * E, E)],
                                      semg[p]).wait()

                def scale_body(gg, _):
                    base = k * E + gg * 16
                    v16 = vv[p, pl.ds(base, 16)]
                    for l in range(16):
                        e = base + l
                        v = v16[l]
                        rbuf[p, e, pl.ds(0, 16)] = rbuf[p, e, pl.ds(0, 16)] * v
                        rbuf[p, e, pl.ds(16, 16)] = (
                            rbuf[p, e, pl.ds(16, 16)] * v)
                    return 0

                lax.fori_loop(0, E // 16, scale_body, 0)
                pltpu.async_copy(rbuf.at[p, pl.ds(k * E, E)],
                                 acc.at[idxr.at[r, k]], sems[p], add=True)
                return 0

            lax.fori_loop(0, KCH, chunk_body, 0)

        def drain_scatter(p, r):
            def drain_body(k, _):
                pltpu.make_async_copy(rbuf.at[p, pl.ds(k * E, E)],
                                      acc.at[idxr.at[r, k]], sems[p]).wait()
                return 0

            lax.fori_loop(0, KCH, drain_body, 0)

        def body(t, p, r, first=False, tail=0):
            if not first:
                drain_scatter((p + 1) % 2, (r + 2) % 3)
            if tail < 2:
                prep(t + 1, (p + 1) % 2, (r + 1) % 3)
            process(t, p, r)
            if tail < 1:
                fire_idx(t + 2, (r + 2) % 3, p)

        fire_idx(0, 0, 0)
        pltpu.sync_copy(zrows, acc.at[pl.ds(sid * RPT, RPT)])
        plsc.subcore_barrier()
        prep(0, 0, 0)
        fire_idx(1, 1, 1)

        body(0, 0, 0, first=True)

        def loop_body(jj, _):
            t = 1 + 6 * jj
            for u in range(6):
                body(t + u, (1 + u) % 2, (1 + u) % 3)
            return 0

        lax.fori_loop(0, 20, loop_body, 0)
        body(121, 1, 1)
        body(122, 0, 2)
        body(123, 1, 0, tail=1)
        body(124, 0, 1, tail=2)
        drain_scatter(0, 1)
        plsc.subcore_barrier()
        pltpu.sync_copy(acc.at[pl.ds(sid * RPT, RPT)],
                        out4.at[g, cid, pl.ds(sid * RPT, RPT)])
        plsc.subcore_barrier()
        return 0

    lax.fori_loop(0, 2, graph_body, 0)


NNP4 = NNP // 4
BN4 = NNP4 // 4


def _update_body(oh_ref, whe_ref, acc_ref, bd_ref, x_out, acc_out, w_out):
    ho0 = oh_ref[0, 0]
    ho1 = oh_ref[0, 1]
    he0 = oh_ref[1, 0]
    he1 = oh_ref[1, 1]
    whe = whe_ref[...]
    who = 1.0 - whe
    a0 = who * ho0 + whe * he0
    a1 = who * ho1 + whe * he1
    bd = bd_ref[...]
    t_ho = jnp.dot(a0 * ho0 + a1 * ho1, bd,
                   preferred_element_type=jnp.float32,
                   precision=lax.Precision.HIGHEST)
    t_he = jnp.dot(a0 * he0 + a1 * he1, bd,
                   preferred_element_type=jnp.float32,
                   precision=lax.Precision.HIGHEST)
    who2 = who + 0.1 * t_ho
    whe2 = whe + 0.1 * t_he
    who3 = who2 / (who2 + whe2)
    w_out[...] = 1.0 - who3
    x_out[0] = a0
    x_out[1] = a1
    acc_out[0] = acc_ref[0] + a0
    acc_out[1] = acc_ref[1] + a1


def _update(out4w, whew, accw, bd):
    big = pl.BlockSpec((2, BN4, 128), lambda i: (0, i, 0))
    big4 = pl.BlockSpec((2, 2, BN4, 128), lambda i: (0, 0, i, 0))
    wide = pl.BlockSpec((BN4, 128), lambda i: (i, 0))
    bspec = pl.BlockSpec((128, 128), lambda i: (0, 0))
    return pl.pallas_call(
        _update_body,
        grid=(NNP4 // BN4,),
        in_specs=[big4, wide, big, bspec],
        out_specs=[big, big, wide],
        out_shape=[
            jax.ShapeDtypeStruct((2, NNP4, 128), jnp.float32),
            jax.ShapeDtypeStruct((2, NNP4, 128), jnp.float32),
            jax.ShapeDtypeStruct((NNP4, 128), jnp.float32),
        ],
    )(out4w, whew, accw, bd)


BPT = B // NTILES
BC = 128


@functools.partial(
    pl.kernel,
    out_type=(
        jax.ShapeDtypeStruct((2, B, DH), jnp.float32),
        jax.ShapeDtypeStruct((2, B, DH), jnp.float32),
    ),
    mesh=_mesh,
    scratch_types=[
        pltpu.VMEM((BC,), jnp.int32),
        pltpu.VMEM((BC,), jnp.int32),
        pltpu.VMEM((BC, DH), jnp.float32),
        pltpu.VMEM((BC, DH), jnp.float32),
        pltpu.SemaphoreType.DMA,
    ],
    compiler_params=pltpu.CompilerParams(use_tc_tiling_on_sc=False),
)
def _gather_rows(accflat, users, items, u_out, i_out, iu, ii, bu, bi, sem):
    cid = lax.axis_index("c")
    sid = lax.axis_index("s")
    coff = cid * NNP
    for q in range(BPT // BC):
        off = sid * BPT + q * BC
        pltpu.sync_copy(users.at[pl.ds(off, BC)], iu)
        pltpu.sync_copy(items.at[pl.ds(off, BC)], ii)
        for i in range(BC // 16):
            iu[pl.ds(i * 16, 16)] = iu[pl.ds(i * 16, 16)] + coff
            ii[pl.ds(i * 16, 16)] = ii[pl.ds(i * 16, 16)] + (coff + NUM_USERS)
        pltpu.async_copy(accflat.at[iu], bu, sem).wait()
        pltpu.async_copy(accflat.at[ii], bi, sem).wait()
        pltpu.sync_copy(bu, u_out.at[cid, pl.ds(off, BC)])
        pltpu.sync_copy(bi, i_out.at[cid, pl.ds(off, BC)])


def _combine_body(u_ref, i_ref, out_ref):
    s = u_ref[0] * i_ref[0] + u_ref[1] * i_ref[1]
    out_ref[...] = jnp.sum(s, axis=1) * 0.0625


def _combine(u2, i2):
    return pl.pallas_call(
        _combine_body,
        out_shape=jax.ShapeDtypeStruct((B,), jnp.float32),
    )(u2, i2)


def kernel(users, items, he_edge_index, he_values, ho_edge_index, ho_values,
           degree_he, user_emb_w, item_emb_w):
    all0 = jnp.concatenate([user_emb_w, item_emb_w], axis=0)
    all0 = jnp.pad(all0, ((0, NNP - NN), (0, 0)))
    x2 = jnp.concatenate([all0[:, :DH], all0[:, DH:]], axis=0)
    accw = jnp.reshape(x2, (2, NNP4, 128))
    whe = jnp.pad(degree_he, ((0, NNP - NN), (0, 0)))
    whew = jnp.reshape(jnp.broadcast_to(whe, (NNP, DH)), (NNP4, 128))
    bd = jnp.kron(jnp.eye(4, dtype=jnp.float32), jnp.ones((32, 32), jnp.float32))
    zrows = jnp.zeros((RPT, DH), jnp.float32)

    rows_all = jnp.reshape(
        jnp.concatenate([ho_edge_index[0], he_edge_index[0]]), (2 * NE // E, E))
    cols_all = jnp.reshape(
        jnp.concatenate([ho_edge_index[1], he_edge_index[1]]), (2 * NE // E, E))
    vals_all = jnp.concatenate([ho_values, he_values])

    for _ in range(NL):
        out4 = _spmm2(x2, rows_all, cols_all, vals_all, zrows)
        out4w = jnp.reshape(out4, (2, 2, NNP4, 128))
        x2w, accw, whew = _update(out4w, whew, accw, bd)
        x2 = jnp.reshape(x2w, (2 * NNP, DH))

    accflat = jnp.reshape(accw, (2 * NNP, DH))
    u2, i2 = _gather_rows(accflat, users, items)
    return _combine(u2, i2)


# speedup vs baseline: 13.6688x; 1.0441x over previous
import functools

import jax
import jax.numpy as jnp
from jax import lax
from jax.experimental import pallas as pl
from jax.experimental.pallas import tpu as pltpu
from jax.experimental.pallas import tpu_sc as plsc

NUM_USERS = 20000
NUM_ITEMS = 30000
NN = NUM_USERS + NUM_ITEMS
NNP = 50048
D = 64
DH = 32
NL = 3
NE = 800000
B = 4096

NTILES = 16
E = 80
KCH = 5
SCH = KCH * E
EPT = NE // NTILES
NT = EPT // SCH
RPT = NNP // NTILES


_mesh = plsc.VectorSubcoreMesh(core_axis_name="c", subcore_axis_name="s")


@functools.partial(
    pl.kernel,
    out_type=jax.ShapeDtypeStruct((2, 2, NNP, DH), jnp.float32),
    mesh=_mesh,
    scratch_types=[
        pltpu.VMEM((3, KCH, E), jnp.int32),
        pltpu.VMEM((3, KCH, E), jnp.int32),
        pltpu.VMEM((2, SCH), jnp.float32),
        pltpu.VMEM((2, SCH, DH), jnp.float32),
        pltpu.VMEM_SHARED((NNP, DH), jnp.float32),
        pltpu.SemaphoreType.DMA,
        pltpu.SemaphoreType.DMA,
        pltpu.SemaphoreType.DMA,
        pltpu.SemaphoreType.DMA,
        pltpu.SemaphoreType.DMA,
        pltpu.SemaphoreType.DMA,
        pltpu.SemaphoreType.DMA,
    ],
    compiler_params=pltpu.CompilerParams(use_tc_tiling_on_sc=False),
)
def _spmm2(x2, ho_rows, ho_cols, ho_vals, he_rows, he_cols, he_vals,
           zrows, out4,
           idxr, idxc, vv, rbuf, acc,
           semi0, semi1, semi2, semg0, semg1, sems0, sems1):
    cid = lax.axis_index("c")
    sid = lax.axis_index("s")
    coff = cid * NNP
    semi = (semi0, semi1, semi2)
    semg = (semg0, semg1)
    sems = (sems0, sems1)

    def graph_body(g, rows_all, cols_all, vals_all):
        base_row = sid * (EPT // E)
        base_e = sid * EPT

        def fire_idx(t, r, p):
            pltpu.async_copy(rows_all.at[pl.ds(base_row + t * KCH, KCH)],
                             idxr.at[r], semi[r])
            pltpu.async_copy(cols_all.at[pl.ds(base_row + t * KCH, KCH)],
                             idxc.at[r], semi[r])
            pltpu.async_copy(vals_all.at[pl.ds(base_e + t * SCH, SCH)],
                             vv.at[p], semi[r])

        def prep(t, p, r):
            pltpu.make_async_copy(rows_all.at[pl.ds(base_row + t * KCH, KCH)],
                                  idxr.at[r], semi[r]).wait()
            pltpu.make_async_copy(cols_all.at[pl.ds(base_row + t * KCH, KCH)],
                                  idxc.at[r], semi[r]).wait()
            pltpu.make_async_copy(vals_all.at[pl.ds(base_e + t * SCH, SCH)],
                                  vv.at[p], semi[r]).wait()

            def off_body(k, _):
                for i in range(E // 16):
                    idxc[r, k, pl.ds(i * 16, 16)] = (
                        idxc[r, k, pl.ds(i * 16, 16)] + coff)
                pltpu.async_copy(x2.at[idxc.at[r, k]],
                                 rbuf.at[p, pl.ds(k * E, E)], semg[p])
                return 0

            lax.fori_loop(0, KCH, off_body, 0)

        def process(t, p, r):
            def chunk_body(k, _):
                pltpu.make_async_copy(x2.at[idxc.at[r, k]],
                                      rbuf.at[p, pl.ds(k * E, E)],
                                      semg[p]).wait()

                def scale_body(gg, _):
                    base = k * E + gg * 16
                    v16 = vv[p, pl.ds(base, 16)]
                    for l in range(16):
                        e = base + l
                        v = v16[l]
                        rbuf[p, e, pl.ds(0, 16)] = rbuf[p, e, pl.ds(0, 16)] * v
                        rbuf[p, e, pl.ds(16, 16)] = (
                            rbuf[p, e, pl.ds(16, 16)] * v)
                    return 0

                lax.fori_loop(0, E // 16, scale_body, 0)
                pltpu.async_copy(rbuf.at[p, pl.ds(k * E, E)],
                                 acc.at[idxr.at[r, k]], sems[p], add=True)
                return 0

            lax.fori_loop(0, KCH, chunk_body, 0)

        def drain_scatter(p, r):
            def drain_body(k, _):
                pltpu.make_async_copy(rbuf.at[p, pl.ds(k * E, E)],
                                      acc.at[idxr.at[r, k]], sems[p]).wait()
                return 0

            lax.fori_loop(0, KCH, drain_body, 0)

        def body(t, p, r, first=False, tail=0):
            if not first:
                drain_scatter((p + 1) % 2, (r + 2) % 3)
            if tail < 2:
                prep(t + 1, (p + 1) % 2, (r + 1) % 3)
            process(t, p, r)
            if tail < 1:
                fire_idx(t + 2, (r + 2) % 3, p)

        fire_idx(0, 0, 0)
        pltpu.sync_copy(zrows, acc.at[pl.ds(sid * RPT, RPT)])
        plsc.subcore_barrier()
        prep(0, 0, 0)
        fire_idx(1, 1, 1)

        body(0, 0, 0, first=True)

        def loop_body(jj, _):
            t = 1 + 6 * jj
            for u in range(6):
                body(t + u, (1 + u) % 2, (1 + u) % 3)
            return 0

        lax.fori_loop(0, 20, loop_body, 0)
        body(121, 1, 1)
        body(122, 0, 2)
        body(123, 1, 0, tail=1)
        body(124, 0, 1, tail=2)
        drain_scatter(0, 1)
        plsc.subcore_barrier()
        pltpu.sync_copy(acc.at[pl.ds(sid * RPT, RPT)],
                        out4.at[g, cid, pl.ds(sid * RPT, RPT)])
        plsc.subcore_barrier()

    graph_body(0, ho_rows, ho_cols, ho_vals)
    graph_body(1, he_rows, he_cols, he_vals)


NNP4 = NNP // 4
BN4 = NNP4 // 4


def _update_body(oh_ref, whe_ref, acc_ref, bd_ref, x_out, acc_out, w_out):
    ho0 = oh_ref[0, 0]
    ho1 = oh_ref[0, 1]
    he0 = oh_ref[1, 0]
    he1 = oh_ref[1, 1]
    whe = whe_ref[...]
    who = 1.0 - whe
    a0 = who * ho0 + whe * he0
    a1 = who * ho1 + whe * he1
    bd = bd_ref[...]
    t_ho = jnp.dot(a0 * ho0 + a1 * ho1, bd,
                   preferred_element_type=jnp.float32,
                   precision=lax.Precision.HIGHEST)
    t_he = jnp.dot(a0 * he0 + a1 * he1, bd,
                   preferred_element_type=jnp.float32,
                   precision=lax.Precision.HIGHEST)
    who2 = who + 0.1 * t_ho
    whe2 = whe + 0.1 * t_he
    who3 = who2 / (who2 + whe2)
    w_out[...] = 1.0 - who3
    x_out[0] = a0
    x_out[1] = a1
    acc_out[0] = acc_ref[0] + a0
    acc_out[1] = acc_ref[1] + a1


def _update(out4w, whew, accw, bd):
    big = pl.BlockSpec((2, BN4, 128), lambda i: (0, i, 0))
    big4 = pl.BlockSpec((2, 2, BN4, 128), lambda i: (0, 0, i, 0))
    wide = pl.BlockSpec((BN4, 128), lambda i: (i, 0))
    bspec = pl.BlockSpec((128, 128), lambda i: (0, 0))
    return pl.pallas_call(
        _update_body,
        grid=(NNP4 // BN4,),
        in_specs=[big4, wide, big, bspec],
        out_specs=[big, big, wide],
        out_shape=[
            jax.ShapeDtypeStruct((2, NNP4, 128), jnp.float32),
            jax.ShapeDtypeStruct((2, NNP4, 128), jnp.float32),
            jax.ShapeDtypeStruct((NNP4, 128), jnp.float32),
        ],
    )(out4w, whew, accw, bd)


BPT = B // NTILES
BC = 128


@functools.partial(
    pl.kernel,
    out_type=(
        jax.ShapeDtypeStruct((2, B, DH), jnp.float32),
        jax.ShapeDtypeStruct((2, B, DH), jnp.float32),
    ),
    mesh=_mesh,
    scratch_types=[
        pltpu.VMEM((BC,), jnp.int32),
        pltpu.VMEM((BC,), jnp.int32),
        pltpu.VMEM((BC, DH), jnp.float32),
        pltpu.VMEM((BC, DH), jnp.float32),
        pltpu.SemaphoreType.DMA,
    ],
    compiler_params=pltpu.CompilerParams(use_tc_tiling_on_sc=False),
)
def _gather_rows(accflat, users, items, u_out, i_out, iu, ii, bu, bi, sem):
    cid = lax.axis_index("c")
    sid = lax.axis_index("s")
    coff = cid * NNP
    for q in range(BPT // BC):
        off = sid * BPT + q * BC
        pltpu.sync_copy(users.at[pl.ds(off, BC)], iu)
        pltpu.sync_copy(items.at[pl.ds(off, BC)], ii)
        for i in range(BC // 16):
            iu[pl.ds(i * 16, 16)] = iu[pl.ds(i * 16, 16)] + coff
            ii[pl.ds(i * 16, 16)] = ii[pl.ds(i * 16, 16)] + (coff + NUM_USERS)
        pltpu.async_copy(accflat.at[iu], bu, sem).wait()
        pltpu.async_copy(accflat.at[ii], bi, sem).wait()
        pltpu.sync_copy(bu, u_out.at[cid, pl.ds(off, BC)])
        pltpu.sync_copy(bi, i_out.at[cid, pl.ds(off, BC)])


def _combine_body(u_ref, i_ref, out_ref):
    s = u_ref[0] * i_ref[0] + u_ref[1] * i_ref[1]
    out_ref[...] = jnp.sum(s, axis=1) * 0.0625


def _combine(u2, i2):
    return pl.pallas_call(
        _combine_body,
        out_shape=jax.ShapeDtypeStruct((B,), jnp.float32),
    )(u2, i2)


def kernel(users, items, he_edge_index, he_values, ho_edge_index, ho_values,
           degree_he, user_emb_w, item_emb_w):
    all0 = jnp.concatenate([user_emb_w, item_emb_w], axis=0)
    all0 = jnp.pad(all0, ((0, NNP - NN), (0, 0)))
    x2 = jnp.concatenate([all0[:, :DH], all0[:, DH:]], axis=0)
    accw = jnp.reshape(x2, (2, NNP4, 128))
    whe = jnp.pad(degree_he, ((0, NNP - NN), (0, 0)))
    whew = jnp.reshape(jnp.broadcast_to(whe, (NNP, DH)), (NNP4, 128))
    bd = jnp.kron(jnp.eye(4, dtype=jnp.float32), jnp.ones((32, 32), jnp.float32))
    zrows = jnp.zeros((RPT, DH), jnp.float32)

    ho_rows = jnp.reshape(ho_edge_index[0], (NE // E, E))
    ho_cols = jnp.reshape(ho_edge_index[1], (NE // E, E))
    he_rows = jnp.reshape(he_edge_index[0], (NE // E, E))
    he_cols = jnp.reshape(he_edge_index[1], (NE // E, E))

    for _ in range(NL):
        out4 = _spmm2(x2, ho_rows, ho_cols, ho_values,
                      he_rows, he_cols, he_values, zrows)
        out4w = jnp.reshape(out4, (2, 2, NNP4, 128))
        x2w, accw, whew = _update(out4w, whew, accw, bd)
        x2 = jnp.reshape(x2w, (2 * NNP, DH))

    accflat = jnp.reshape(accw, (2 * NNP, DH))
    u2, i2 = _gather_rows(accflat, users, items)
    return _combine(u2, i2)
